# baseline (speedup 1.0000x reference)
import functools

import jax
import jax.numpy as jnp
from jax import lax
from jax.experimental import pallas as pl
from jax.experimental.pallas import tpu as pltpu
from jax.experimental.pallas import tpu_sc as plsc

N = 10000
E = 320000
D = 128

NC = 2
NS = 16
CHUNK = 128
SUP = 8
NCH = 160
NSUP = NCH // SUP
PER_TILE = NCH * CHUNK
E_PAD = PER_TILE * NS
ROWS_PER_TILE = 640
N_PAD = ROWS_PER_TILE * NS

PRO_BLK = 400
EPI_BLK = 512
NT_PAD = 1024


def _pro_body(ids_ref, m_ref, x_ref, meff_ref, mx_ref):
    base = pl.program_id(0) * PRO_BLK
    rows = base + lax.broadcasted_iota(jnp.int32, (PRO_BLK, 1), 0)
    ids = ids_ref[...]
    hit = jnp.zeros((PRO_BLK, 1), dtype=jnp.bool_)
    for j in range(NT_PAD // 128):
        hit = hit | jnp.any(rows == ids[j:j + 1, :], axis=1, keepdims=True)
    meff = jnp.where(hit, 1.0, jax.nn.sigmoid(m_ref[...]))
    meff_ref[...] = meff
    mx_ref[...] = meff * x_ref[...]


def _prologue(train_ids_pad, m, x):
    return pl.pallas_call(
        _pro_body,
        grid=(N // PRO_BLK,),
        in_specs=[
            pl.BlockSpec((NT_PAD // 128, 128), lambda i: (0, 0)),
            pl.BlockSpec((PRO_BLK, D), lambda i: (i, 0)),
            pl.BlockSpec((PRO_BLK, D), lambda i: (i, 0)),
        ],
        out_specs=[
            pl.BlockSpec((PRO_BLK, D), lambda i: (i, 0)),
            pl.BlockSpec((PRO_BLK, D), lambda i: (i, 0)),
        ],
        out_shape=[
            jax.ShapeDtypeStruct((N, D), jnp.float32),
            jax.ShapeDtypeStruct((N, D), jnp.float32),
        ],
    )(train_ids_pad, m, x)


def _sc_body(meff_hbm, mx_hbm, sidx_hbm, didx_hbm, w_hbm, dsum_hbm, hsum_hbm,
             sb_s0, sb_s1, sb_d0, sb_d1, sb_w0, sb_w1, didx0, didx1,
             rows0, rows1, acc_sh, gsem0, gsem1, ssem0, ssem1, stsem):
    cid = lax.axis_index("c")
    tid = lax.axis_index("s")
    tbase = tid * PER_TILE
    sb_s = (sb_s0, sb_s1)
    sb_d = (sb_d0, sb_d1)
    sb_w = (sb_w0, sb_w1)
    didx = (didx0, didx1)
    rows = (rows0, rows1)
    gsem = (gsem0, gsem1)
    ssem = (ssem0, ssem1)

    def _zero_row(i, _):
        for j in range(D // 16):
            rows0[i, pl.ds(j * 16, 16)] = jnp.zeros((16,), jnp.float32)
        return 0
    lax.fori_loop(0, CHUNK, _zero_row, 0)
    for k in range(ROWS_PER_TILE // CHUNK):
        pltpu.sync_copy(
            rows0, acc_sh.at[pl.ds(tid * ROWS_PER_TILE + k * CHUNK, CHUNK)])
    plsc.subcore_barrier()

    SLEN = SUP * CHUNK

    def _run(scaled):
        table = mx_hbm if scaled else meff_hbm

        def _stage(s, sb, sync):
            off = tbase + s * SLEN
            if sync:
                pltpu.sync_copy(sidx_hbm.at[pl.ds(off, SLEN)], sb_s[sb])
                pltpu.sync_copy(didx_hbm.at[pl.ds(off, SLEN)], sb_d[sb])
                if scaled:
                    pltpu.sync_copy(w_hbm.at[pl.ds(off, SLEN)], sb_w[sb])
            else:
                pltpu.async_copy(sidx_hbm.at[pl.ds(off, SLEN)], sb_s[sb],
                                 stsem)
                pltpu.async_copy(didx_hbm.at[pl.ds(off, SLEN)], sb_d[sb],
                                 stsem)
                if scaled:
                    pltpu.async_copy(w_hbm.at[pl.ds(off, SLEN)], sb_w[sb],
                                     stsem)

        def _wait_stage(sb):
            pltpu.make_async_copy(sidx_hbm.at[pl.ds(0, SLEN)], sb_s[sb],
                                  stsem).wait()
            pltpu.make_async_copy(didx_hbm.at[pl.ds(0, SLEN)], sb_d[sb],
                                  stsem).wait()
            if scaled:
                pltpu.make_async_copy(w_hbm.at[pl.ds(0, SLEN)], sb_w[sb],
                                      stsem).wait()

        def _issue(k, sb, b):
            for j in range(CHUNK // 16):
                didx[b][pl.ds(j * 16, 16)] = (
                    sb_d[sb][pl.ds(k * CHUNK + j * 16, 16)])
            pltpu.async_copy(
                table.at[sb_s[sb].at[pl.ds(k * CHUNK, CHUNK)]], rows[b],
                gsem[b])

        def _step(s, p, k, scaled):
            b = k % 2
            nxt = b ^ 1

            def _wait_prev():
                pltpu.make_async_copy(
                    rows[nxt], acc_sh.at[didx[nxt]], ssem[nxt]).wait()

            if k == 0 and p == 0:
                @pl.when(s >= 1)
                def _():
                    _wait_prev()
            else:
                _wait_prev()

            if k + 1 < SUP:
                _issue(k + 1, p, nxt)
            else:
                @pl.when(s + 1 < NSUP)
                def _():
                    _wait_stage(p ^ 1)
                    _issue(0, p ^ 1, nxt)

            pltpu.make_async_copy(
                table.at[sb_s[p].at[pl.ds(k * CHUNK, CHUNK)]], rows[b],
                gsem[b]).wait()
            if scaled:
                def _scale_grp(g, _):
                    wv = sb_w[p][pl.ds(k * CHUNK + g * 16, 16)]
                    for l in range(16):
                        wl = wv[l]
                        e = g * 16 + l
                        for j in range(D // 16):
                            sl = pl.ds(j * 16, 16)
                            rows[b][e, sl] = rows[b][e, sl] * wl
                    return 0
                lax.fori_loop(0, CHUNK // 16, _scale_grp, 0)
            pltpu.async_copy(rows[b], acc_sh.at[didx[b]], ssem[b], add=True)

        _stage(0, 0, sync=True)
        _issue(0, 0, 0)

        def _super(h, _):
            for p in range(2):
                s = h * 2 + p

                @pl.when(s + 1 < NSUP)
                def _():
                    _stage(s + 1, p ^ 1, sync=False)
                for k in range(SUP):
                    _step(s, p, k, scaled)
            return 0
        lax.fori_loop(0, NSUP // 2, _super, 0)
        pltpu.make_async_copy(rows[1], acc_sh.at[didx[1]], ssem[1]).wait()

    @pl.when(cid == 0)
    def _():
        _run(scaled=False)

    @pl.when(cid == 1)
    def _():
        _run(scaled=True)

    plsc.subcore_barrier()

    out_slice = pl.ds(tid * ROWS_PER_TILE, ROWS_PER_TILE)

    @pl.when(cid == 0)
    def _():
        pltpu.sync_copy(acc_sh.at[out_slice], dsum_hbm.at[out_slice])

    @pl.when(cid == 1)
    def _():
        pltpu.sync_copy(acc_sh.at[out_slice], hsum_hbm.at[out_slice])


def _segment_sums(meff, mx, sidx, didx, w):
    f32 = jnp.float32
    kern = pl.kernel(
        _sc_body,
        out_type=[
            jax.ShapeDtypeStruct((N_PAD, D), f32),
            jax.ShapeDtypeStruct((N_PAD, D), f32),
        ],
        mesh=plsc.VectorSubcoreMesh(core_axis_name="c", subcore_axis_name="s"),
        scratch_types=[
            pltpu.VMEM((SUP * CHUNK,), jnp.int32),
            pltpu.VMEM((SUP * CHUNK,), jnp.int32),
            pltpu.VMEM((SUP * CHUNK,), jnp.int32),
            pltpu.VMEM((SUP * CHUNK,), jnp.int32),
            pltpu.VMEM((SUP * CHUNK,), f32),
            pltpu.VMEM((SUP * CHUNK,), f32),
            pltpu.VMEM((CHUNK,), jnp.int32),
            pltpu.VMEM((CHUNK,), jnp.int32),
            pltpu.VMEM((CHUNK, D), f32),
            pltpu.VMEM((CHUNK, D), f32),
            pltpu.VMEM_SHARED((N_PAD, D), f32),
            pltpu.SemaphoreType.DMA,
            pltpu.SemaphoreType.DMA,
            pltpu.SemaphoreType.DMA,
            pltpu.SemaphoreType.DMA,
            pltpu.SemaphoreType.DMA,
        ],
    )
    return kern(meff, mx, sidx, didx, w)


def _epi_body(d_ref, h_ref, w_ref, out_ref):
    d = d_ref[...]
    am = jnp.where(d == 0.0, 0.0, 1.0 / d)
    h = h_ref[...] * am
    p = jnp.dot(h, w_ref[...], preferred_element_type=jnp.float32)
    out_ref[...] = jnp.where(p > 0.0, p, jnp.exp(p) - 1.0)


def _epilogue(dsum, hsum, w):
    return pl.pallas_call(
        _epi_body,
        grid=(N_PAD // EPI_BLK,),
        in_specs=[
            pl.BlockSpec((EPI_BLK, D), lambda i: (i, 0)),
            pl.BlockSpec((EPI_BLK, D), lambda i: (i, 0)),
            pl.BlockSpec((D, D), lambda i: (0, 0)),
        ],
        out_specs=pl.BlockSpec((EPI_BLK, D), lambda i: (i, 0)),
        out_shape=jax.ShapeDtypeStruct((N_PAD, D), jnp.float32),
    )(dsum, hsum, w)


@jax.jit
def kernel(x, edge_index, edge_weight, train_fts_id, W, M):
    src = edge_index[0].astype(jnp.int32)
    dst = edge_index[1].astype(jnp.int32)
    w = edge_weight.astype(jnp.float32)

    pad = E_PAD - E
    sidx = jnp.concatenate([src, jnp.zeros((pad,), jnp.int32)])
    didx = jnp.concatenate([dst, jnp.full((pad,), N, jnp.int32)])
    wpad = jnp.concatenate([w, jnp.zeros((pad,), jnp.float32)])

    ids = train_fts_id.astype(jnp.int32)
    ids_pad = jnp.concatenate(
        [ids, jnp.full((NT_PAD - ids.shape[0],), -1, jnp.int32)]
    ).reshape(NT_PAD // 128, 128)

    meff, mx = _prologue(ids_pad, M, x)
    dsum, hsum = _segment_sums(meff, mx, sidx, didx, wpad)
    out = _epilogue(dsum, hsum, W)
    return out[:N]


# speedup vs baseline: 1.1280x; 1.1280x over previous
import functools

import jax
import jax.numpy as jnp
from jax import lax
from jax.experimental import pallas as pl
from jax.experimental.pallas import tpu as pltpu
from jax.experimental.pallas import tpu_sc as plsc

N = 10000
E = 320000
D = 128

NC = 2
NS = 16
CHUNK = 128
SUP = 8
NCH = 160
NSUP = NCH // SUP
PER_TILE = NCH * CHUNK
E_PAD = PER_TILE * NS
ROWS_PER_TILE = 640
N_PAD = ROWS_PER_TILE * NS

PRO_BLK = 400
EPI_BLK = 512
NT_PAD = 1024


def _pro_body(ids_ref, m_ref, x_ref, meff_ref, mx_ref):
    base = pl.program_id(0) * PRO_BLK
    rows = base + lax.broadcasted_iota(jnp.int32, (PRO_BLK, 1), 0)
    ids = ids_ref[...]
    hit = jnp.zeros((PRO_BLK, 1), dtype=jnp.bool_)
    for j in range(NT_PAD // 128):
        hit = hit | jnp.any(rows == ids[j:j + 1, :], axis=1, keepdims=True)
    meff = jnp.where(hit, 1.0, jax.nn.sigmoid(m_ref[...]))
    meff_ref[...] = meff
    mx_ref[...] = meff * x_ref[...]


def _prologue(train_ids_pad, m, x):
    return pl.pallas_call(
        _pro_body,
        grid=(N // PRO_BLK,),
        in_specs=[
            pl.BlockSpec((NT_PAD // 128, 128), lambda i: (0, 0)),
            pl.BlockSpec((PRO_BLK, D), lambda i: (i, 0)),
            pl.BlockSpec((PRO_BLK, D), lambda i: (i, 0)),
        ],
        out_specs=[
            pl.BlockSpec((PRO_BLK, D), lambda i: (i, 0)),
            pl.BlockSpec((PRO_BLK, D), lambda i: (i, 0)),
        ],
        out_shape=[
            jax.ShapeDtypeStruct((N, D), jnp.float32),
            jax.ShapeDtypeStruct((N, D), jnp.float32),
        ],
    )(train_ids_pad, m, x)


def _sc_body(meff_hbm, mx_hbm, sidx_hbm, didx_hbm, w_hbm, dsum_hbm, hsum_hbm,
             sb_s0, sb_s1, sb_d0, sb_d1, sb_w0, sb_w1,
             rows0, rows1, acc_sh, gsem0, gsem1, ssem0, ssem1, stsem):
    cid = lax.axis_index("c")
    tid = lax.axis_index("s")
    sb_s = (sb_s0, sb_s1)
    sb_d = (sb_d0, sb_d1)
    sb_w = (sb_w0, sb_w1)
    rows = (rows0, rows1)
    gsem = (gsem0, gsem1)
    ssem = (ssem0, ssem1)

    def _zero_row(i, _):
        for j in range(D // 16):
            rows0[i, pl.ds(j * 16, 16)] = jnp.zeros((16,), jnp.float32)
        return 0
    lax.fori_loop(0, CHUNK, _zero_row, 0)
    for k in range(ROWS_PER_TILE // CHUNK):
        pltpu.sync_copy(
            rows0, acc_sh.at[pl.ds(tid * ROWS_PER_TILE + k * CHUNK, CHUNK)])
    plsc.subcore_barrier()

    def _run(scaled):
        table = mx_hbm if scaled else meff_hbm

        def _stage(s, sb, sync):
            blk = tid * NSUP + s
            if sync:
                pltpu.sync_copy(sidx_hbm.at[blk], sb_s[sb])
                pltpu.sync_copy(didx_hbm.at[blk], sb_d[sb])
                if scaled:
                    pltpu.sync_copy(w_hbm.at[blk], sb_w[sb])
            else:
                pltpu.async_copy(sidx_hbm.at[blk], sb_s[sb], stsem)
                pltpu.async_copy(didx_hbm.at[blk], sb_d[sb], stsem)
                if scaled:
                    pltpu.async_copy(w_hbm.at[blk], sb_w[sb], stsem)

        def _wait_stage(sb):
            pltpu.make_async_copy(sidx_hbm.at[0], sb_s[sb], stsem).wait()
            pltpu.make_async_copy(didx_hbm.at[0], sb_d[sb], stsem).wait()
            if scaled:
                pltpu.make_async_copy(w_hbm.at[0], sb_w[sb], stsem).wait()

        def _issue(k, sb, b):
            pltpu.async_copy(table.at[sb_s[sb].at[k]], rows[b], gsem[b])

        def _wait_scat(b):
            pltpu.make_async_copy(
                rows[b], acc_sh.at[sb_d[0].at[0]], ssem[b]).wait()

        def _step(s, p, k, scaled):
            b = k % 2
            nxt = b ^ 1

            if k == 0 and p == 0:
                @pl.when(s >= 1)
                def _():
                    _wait_scat(nxt)
            else:
                _wait_scat(nxt)

            if k + 1 < SUP:
                _issue(k + 1, p, nxt)
            else:
                @pl.when(s + 1 < NSUP)
                def _():
                    _wait_stage(p ^ 1)
                    _issue(0, p ^ 1, nxt)

            pltpu.make_async_copy(
                table.at[sb_s[p].at[k]], rows[b], gsem[b]).wait()
            if scaled:
                def _scale_grp(g, _):
                    wv = sb_w[p][pl.ds(k * CHUNK + g * 16, 16)]
                    for l in range(16):
                        wl = wv[l]
                        e = g * 16 + l
                        for j in range(D // 16):
                            sl = pl.ds(j * 16, 16)
                            rows[b][e, sl] = rows[b][e, sl] * wl
                    return 0
                lax.fori_loop(0, CHUNK // 16, _scale_grp, 0)
            pltpu.async_copy(rows[b], acc_sh.at[sb_d[p].at[k]], ssem[b],
                             add=True)

        _stage(0, 0, sync=True)
        _issue(0, 0, 0)

        def _super(h, _):
            for p in range(2):
                s = h * 2 + p

                @pl.when(s + 1 < NSUP)
                def _():
                    _stage(s + 1, p ^ 1, sync=False)
                for k in range(SUP):
                    _step(s, p, k, scaled)
            return 0
        lax.fori_loop(0, NSUP // 2, _super, 0)
        _wait_scat(1)

    @pl.when(cid == 0)
    def _():
        _run(scaled=False)

    @pl.when(cid == 1)
    def _():
        _run(scaled=True)

    plsc.subcore_barrier()

    out_slice = pl.ds(tid * ROWS_PER_TILE, ROWS_PER_TILE)

    @pl.when(cid == 0)
    def _():
        pltpu.sync_copy(acc_sh.at[out_slice], dsum_hbm.at[out_slice])

    @pl.when(cid == 1)
    def _():
        pltpu.sync_copy(acc_sh.at[out_slice], hsum_hbm.at[out_slice])


def _segment_sums(meff, mx, sidx, didx, w):
    f32 = jnp.float32
    kern = pl.kernel(
        _sc_body,
        out_type=[
            jax.ShapeDtypeStruct((N_PAD, D), f32),
            jax.ShapeDtypeStruct((N_PAD, D), f32),
        ],
        mesh=plsc.VectorSubcoreMesh(core_axis_name="c", subcore_axis_name="s"),
        scratch_types=[
            pltpu.VMEM((SUP, CHUNK), jnp.int32),
            pltpu.VMEM((SUP, CHUNK), jnp.int32),
            pltpu.VMEM((SUP, CHUNK), jnp.int32),
            pltpu.VMEM((SUP, CHUNK), jnp.int32),
            pltpu.VMEM((SUP * CHUNK,), f32),
            pltpu.VMEM((SUP * CHUNK,), f32),
            pltpu.VMEM((CHUNK, D), f32),
            pltpu.VMEM((CHUNK, D), f32),
            pltpu.VMEM_SHARED((N_PAD, D), f32),
            pltpu.SemaphoreType.DMA,
            pltpu.SemaphoreType.DMA,
            pltpu.SemaphoreType.DMA,
            pltpu.SemaphoreType.DMA,
            pltpu.SemaphoreType.DMA,
        ],
    )
    return kern(meff, mx, sidx, didx, w)


def _epi_body(d_ref, h_ref, w_ref, out_ref):
    d = d_ref[...]
    am = jnp.where(d == 0.0, 0.0, 1.0 / d)
    h = h_ref[...] * am
    p = jnp.dot(h, w_ref[...], preferred_element_type=jnp.float32)
    out_ref[...] = jnp.where(p > 0.0, p, jnp.exp(p) - 1.0)


def _epilogue(dsum, hsum, w):
    return pl.pallas_call(
        _epi_body,
        grid=(N_PAD // EPI_BLK,),
        in_specs=[
            pl.BlockSpec((EPI_BLK, D), lambda i: (i, 0)),
            pl.BlockSpec((EPI_BLK, D), lambda i: (i, 0)),
            pl.BlockSpec((D, D), lambda i: (0, 0)),
        ],
        out_specs=pl.BlockSpec((EPI_BLK, D), lambda i: (i, 0)),
        out_shape=jax.ShapeDtypeStruct((N_PAD, D), jnp.float32),
    )(dsum, hsum, w)


@jax.jit
def kernel(x, edge_index, edge_weight, train_fts_id, W, M):
    src = edge_index[0].astype(jnp.int32)
    dst = edge_index[1].astype(jnp.int32)
    w = edge_weight.astype(jnp.float32)

    pad = E_PAD - E
    sidx = jnp.concatenate([src, jnp.zeros((pad,), jnp.int32)])
    didx = jnp.concatenate([dst, jnp.full((pad,), N, jnp.int32)])
    wpad = jnp.concatenate([w, jnp.zeros((pad,), jnp.float32)])
    sidx = sidx.reshape(NS * NSUP, SUP, CHUNK)
    didx = didx.reshape(NS * NSUP, SUP, CHUNK)
    wpad = wpad.reshape(NS * NSUP, SUP * CHUNK)

    ids = train_fts_id.astype(jnp.int32)
    ids_pad = jnp.concatenate(
        [ids, jnp.full((NT_PAD - ids.shape[0],), -1, jnp.int32)]
    ).reshape(NT_PAD // 128, 128)

    meff, mx = _prologue(ids_pad, M, x)
    dsum, hsum = _segment_sums(meff, mx, sidx, didx, wpad)
    out = _epilogue(dsum, hsum, W)
    return out[:N]


# speedup vs baseline: 1.6329x; 1.4475x over previous
import functools

import jax
import jax.numpy as jnp
from jax import lax
from jax.experimental import pallas as pl
from jax.experimental.pallas import tpu as pltpu
from jax.experimental.pallas import tpu_sc as plsc

N = 10000
E = 320000
D = 128

NC = 2
NS = 16
CHUNK = 112
SUP = 6
NBUF = 3
NCH = 180
NSUP = NCH // SUP
PER_TILE = NCH * CHUNK
E_PAD = PER_TILE * NS
ROWS_PER_TILE = 640
N_PAD = ROWS_PER_TILE * NS

PRO_BLK = 400
EPI_BLK = 512
NT_PAD = 1024


def _pro_body(ids_ref, m_ref, x_ref, meff_ref, mx_ref):
    base = pl.program_id(0) * PRO_BLK
    rows = base + lax.broadcasted_iota(jnp.int32, (PRO_BLK, 1), 0)
    ids = ids_ref[...]
    hit = jnp.zeros((PRO_BLK, 1), dtype=jnp.bool_)
    for j in range(NT_PAD // 128):
        hit = hit | jnp.any(rows == ids[j:j + 1, :], axis=1, keepdims=True)
    meff = jnp.where(hit, 1.0, jax.nn.sigmoid(m_ref[...]))
    meff_ref[...] = meff
    mx_ref[...] = meff * x_ref[...]


def _prologue(train_ids_pad, m, x):
    return pl.pallas_call(
        _pro_body,
        grid=(N // PRO_BLK,),
        in_specs=[
            pl.BlockSpec((NT_PAD // 128, 128), lambda i: (0, 0)),
            pl.BlockSpec((PRO_BLK, D), lambda i: (i, 0)),
            pl.BlockSpec((PRO_BLK, D), lambda i: (i, 0)),
        ],
        out_specs=[
            pl.BlockSpec((PRO_BLK, D), lambda i: (i, 0)),
            pl.BlockSpec((PRO_BLK, D), lambda i: (i, 0)),
        ],
        out_shape=[
            jax.ShapeDtypeStruct((N, D), jnp.float32),
            jax.ShapeDtypeStruct((N, D), jnp.float32),
        ],
    )(train_ids_pad, m, x)


def _sc_body(meff_hbm, mx_hbm, sidx_hbm, didx_hbm, w_hbm, dsum_hbm, hsum_hbm,
             sb_s0, sb_s1, sb_d0, sb_d1, sb_w0, sb_w1,
             rows0, rows1, rows2, acc_sh,
             gsem0, gsem1, gsem2, ssem0, ssem1, ssem2, stsem):
    cid = lax.axis_index("c")
    tid = lax.axis_index("s")
    sb_s = (sb_s0, sb_s1)
    sb_d = (sb_d0, sb_d1)
    sb_w = (sb_w0, sb_w1)
    rows = (rows0, rows1, rows2)
    gsem = (gsem0, gsem1, gsem2)
    ssem = (ssem0, ssem1, ssem2)

    def _zero_row(i, _):
        for j in range(D // 16):
            rows0[i, pl.ds(j * 16, 16)] = jnp.zeros((16,), jnp.float32)
        return 0
    lax.fori_loop(0, CHUNK, _zero_row, 0)
    zbase = tid * ROWS_PER_TILE
    for k in range(ROWS_PER_TILE // CHUNK):
        pltpu.sync_copy(
            rows0, acc_sh.at[pl.ds(zbase + k * CHUNK, CHUNK)])
    rem = ROWS_PER_TILE % CHUNK
    if rem:
        pltpu.sync_copy(
            rows0.at[pl.ds(0, rem)],
            acc_sh.at[pl.ds(zbase + ROWS_PER_TILE - rem, rem)])
    plsc.subcore_barrier()

    def _run(scaled):
        table = mx_hbm if scaled else meff_hbm

        def _stage(s, sb, sync):
            blk = tid * NSUP + s
            if sync:
                pltpu.sync_copy(sidx_hbm.at[blk], sb_s[sb])
                pltpu.sync_copy(didx_hbm.at[blk], sb_d[sb])
                if scaled:
                    pltpu.sync_copy(w_hbm.at[blk], sb_w[sb])
            else:
                pltpu.async_copy(sidx_hbm.at[blk], sb_s[sb], stsem)
                pltpu.async_copy(didx_hbm.at[blk], sb_d[sb], stsem)
                if scaled:
                    pltpu.async_copy(w_hbm.at[blk], sb_w[sb], stsem)

        def _wait_stage(sb):
            pltpu.make_async_copy(sidx_hbm.at[0], sb_s[sb], stsem).wait()
            pltpu.make_async_copy(didx_hbm.at[0], sb_d[sb], stsem).wait()
            if scaled:
                pltpu.make_async_copy(w_hbm.at[0], sb_w[sb], stsem).wait()

        def _issue(k, sb, b):
            pltpu.async_copy(table.at[sb_s[sb].at[k]], rows[b], gsem[b])

        def _wait_scat(b):
            pltpu.make_async_copy(
                rows[b], acc_sh.at[sb_d[0].at[0]], ssem[b]).wait()

        def _step(s, p, k, scaled):
            b = k % NBUF
            nxt = (k + 1) % NBUF

            if k <= 1 and p == 0:
                @pl.when(s >= 1)
                def _():
                    _wait_scat(nxt)
            else:
                _wait_scat(nxt)

            if k + 1 < SUP:
                _issue(k + 1, p, nxt)
            else:
                @pl.when(s + 1 < NSUP)
                def _():
                    _wait_stage(p ^ 1)
                    _issue(0, p ^ 1, nxt)

            pltpu.make_async_copy(
                table.at[sb_s[p].at[k]], rows[b], gsem[b]).wait()
            if scaled:
                def _scale_grp(g, _):
                    wv = sb_w[p][pl.ds(k * CHUNK + g * 16, 16)]
                    for l in range(16):
                        wl = wv[l]
                        e = g * 16 + l
                        for j in range(D // 16):
                            sl = pl.ds(j * 16, 16)
                            rows[b][e, sl] = rows[b][e, sl] * wl
                    return 0
                lax.fori_loop(0, CHUNK // 16, _scale_grp, 0)
            pltpu.async_copy(rows[b], acc_sh.at[sb_d[p].at[k]], ssem[b],
                             add=True)

        _stage(0, 0, sync=True)
        _issue(0, 0, 0)

        def _super(h, _):
            for p in range(2):
                s = h * 2 + p

                @pl.when(s + 1 < NSUP)
                def _():
                    _stage(s + 1, p ^ 1, sync=False)
                for k in range(SUP):
                    _step(s, p, k, scaled)
            return 0
        lax.fori_loop(0, NSUP // 2, _super, 0)
        _wait_scat((NCH - 2) % NBUF)
        _wait_scat((NCH - 1) % NBUF)

    @pl.when(cid == 0)
    def _():
        _run(scaled=False)

    @pl.when(cid == 1)
    def _():
        _run(scaled=True)

    plsc.subcore_barrier()

    out_slice = pl.ds(tid * ROWS_PER_TILE, ROWS_PER_TILE)

    @pl.when(cid == 0)
    def _():
        pltpu.sync_copy(acc_sh.at[out_slice], dsum_hbm.at[out_slice])

    @pl.when(cid == 1)
    def _():
        pltpu.sync_copy(acc_sh.at[out_slice], hsum_hbm.at[out_slice])


def _segment_sums(meff, mx, sidx, didx, w):
    f32 = jnp.float32
    kern = pl.kernel(
        _sc_body,
        out_type=[
            jax.ShapeDtypeStruct((N_PAD, D), f32),
            jax.ShapeDtypeStruct((N_PAD, D), f32),
        ],
        mesh=plsc.VectorSubcoreMesh(core_axis_name="c", subcore_axis_name="s"),
        scratch_types=[
            pltpu.VMEM((SUP, CHUNK), jnp.int32),
            pltpu.VMEM((SUP, CHUNK), jnp.int32),
            pltpu.VMEM((SUP, CHUNK), jnp.int32),
            pltpu.VMEM((SUP, CHUNK), jnp.int32),
            pltpu.VMEM((SUP * CHUNK,), f32),
            pltpu.VMEM((SUP * CHUNK,), f32),
            pltpu.VMEM((CHUNK, D), f32),
            pltpu.VMEM((CHUNK, D), f32),
            pltpu.VMEM((CHUNK, D), f32),
            pltpu.VMEM_SHARED((N_PAD, D), f32),
            pltpu.SemaphoreType.DMA,
            pltpu.SemaphoreType.DMA,
            pltpu.SemaphoreType.DMA,
            pltpu.SemaphoreType.DMA,
            pltpu.SemaphoreType.DMA,
            pltpu.SemaphoreType.DMA,
            pltpu.SemaphoreType.DMA,
        ],
    )
    return kern(meff, mx, sidx, didx, w)


def _epi_body(d_ref, h_ref, w_ref, out_ref):
    d = d_ref[...]
    am = jnp.where(d == 0.0, 0.0, 1.0 / d)
    h = h_ref[...] * am
    p = jnp.dot(h, w_ref[...], preferred_element_type=jnp.float32)
    out_ref[...] = jnp.where(p > 0.0, p, jnp.exp(p) - 1.0)


def _epilogue(dsum, hsum, w):
    return pl.pallas_call(
        _epi_body,
        grid=(N_PAD // EPI_BLK,),
        in_specs=[
            pl.BlockSpec((EPI_BLK, D), lambda i: (i, 0)),
            pl.BlockSpec((EPI_BLK, D), lambda i: (i, 0)),
            pl.BlockSpec((D, D), lambda i: (0, 0)),
        ],
        out_specs=pl.BlockSpec((EPI_BLK, D), lambda i: (i, 0)),
        out_shape=jax.ShapeDtypeStruct((N_PAD, D), jnp.float32),
    )(dsum, hsum, w)


@jax.jit
def kernel(x, edge_index, edge_weight, train_fts_id, W, M):
    src = edge_index[0].astype(jnp.int32)
    dst = edge_index[1].astype(jnp.int32)
    w = edge_weight.astype(jnp.float32)

    pad = E_PAD - E
    sidx = jnp.concatenate([src, jnp.zeros((pad,), jnp.int32)])
    didx = jnp.concatenate([dst, jnp.full((pad,), N, jnp.int32)])
    wpad = jnp.concatenate([w, jnp.zeros((pad,), jnp.float32)])
    sidx = sidx.reshape(NS * NSUP, SUP, CHUNK)
    didx = didx.reshape(NS * NSUP, SUP, CHUNK)
    wpad = wpad.reshape(NS * NSUP, SUP * CHUNK)

    ids = train_fts_id.astype(jnp.int32)
    ids_pad = jnp.concatenate(
        [ids, jnp.full((NT_PAD - ids.shape[0],), -1, jnp.int32)]
    ).reshape(NT_PAD // 128, 128)

    meff, mx = _prologue(ids_pad, M, x)
    dsum, hsum = _segment_sums(meff, mx, sidx, didx, wpad)
    out = _epilogue(dsum, hsum, W)
    return out[:N]


# speedup vs baseline: 1.6813x; 1.0297x over previous
import functools

import jax
import jax.numpy as jnp
from jax import lax
from jax.experimental import pallas as pl
from jax.experimental.pallas import tpu as pltpu
from jax.experimental.pallas import tpu_sc as plsc

N = 10000
E = 320000
D = 128

NC = 2
NS = 16
CHUNK = 112
SUP = 6
NBUF = 3
NCH = 180
NSUP = NCH // SUP
PER_TILE = NCH * CHUNK
E_PAD = PER_TILE * NS
ROWS_PER_TILE = 640
N_PAD = ROWS_PER_TILE * NS

PRO_BLK = 400
EPI_BLK = 512
NT_PAD = 1024


def _pro_body(ids_ref, m_ref, x_ref, meff_ref, mx_ref):
    base = pl.program_id(0) * PRO_BLK
    rows = base + lax.broadcasted_iota(jnp.int32, (PRO_BLK, 1), 0)
    ids = ids_ref[...]
    hit = jnp.zeros((PRO_BLK, 1), dtype=jnp.bool_)
    for j in range(NT_PAD // 128):
        hit = hit | jnp.any(rows == ids[j:j + 1, :], axis=1, keepdims=True)
    meff = jnp.where(hit, 1.0, jax.nn.sigmoid(m_ref[...]))
    mx = meff * x_ref[...]
    meff_ref[...] = jnp.concatenate([meff[:, :D // 2], mx[:, :D // 2]], 1)
    mx_ref[...] = jnp.concatenate([meff[:, D // 2:], mx[:, D // 2:]], 1)


def _prologue(train_ids_pad, m, x):
    return pl.pallas_call(
        _pro_body,
        grid=(N // PRO_BLK,),
        in_specs=[
            pl.BlockSpec((NT_PAD // 128, 128), lambda i: (0, 0)),
            pl.BlockSpec((PRO_BLK, D), lambda i: (i, 0)),
            pl.BlockSpec((PRO_BLK, D), lambda i: (i, 0)),
        ],
        out_specs=[
            pl.BlockSpec((PRO_BLK, D), lambda i: (i, 0)),
            pl.BlockSpec((PRO_BLK, D), lambda i: (i, 0)),
        ],
        out_shape=[
            jax.ShapeDtypeStruct((N, D), jnp.float32),
            jax.ShapeDtypeStruct((N, D), jnp.float32),
        ],
    )(train_ids_pad, m, x)


def _sc_body(meff_hbm, mx_hbm, sidx_hbm, didx_hbm, w_hbm, dsum_hbm, hsum_hbm,
             sb_s0, sb_s1, sb_d0, sb_d1, sb_w0, sb_w1,
             rows0, rows1, rows2, acc_sh,
             gsem0, gsem1, gsem2, ssem0, ssem1, ssem2, stsem):
    cid = lax.axis_index("c")
    tid = lax.axis_index("s")
    sb_s = (sb_s0, sb_s1)
    sb_d = (sb_d0, sb_d1)
    sb_w = (sb_w0, sb_w1)
    rows = (rows0, rows1, rows2)
    gsem = (gsem0, gsem1, gsem2)
    ssem = (ssem0, ssem1, ssem2)

    def _zero_row(i, _):
        for j in range(D // 16):
            rows0[i, pl.ds(j * 16, 16)] = jnp.zeros((16,), jnp.float32)
        return 0
    lax.fori_loop(0, CHUNK, _zero_row, 0)
    zbase = tid * ROWS_PER_TILE
    for k in range(ROWS_PER_TILE // CHUNK):
        pltpu.sync_copy(
            rows0, acc_sh.at[pl.ds(zbase + k * CHUNK, CHUNK)])
    rem = ROWS_PER_TILE % CHUNK
    if rem:
        pltpu.sync_copy(
            rows0.at[pl.ds(0, rem)],
            acc_sh.at[pl.ds(zbase + ROWS_PER_TILE - rem, rem)])
    plsc.subcore_barrier()

    def _run(table):
        def _stage(s, sb, sync):
            blk = tid * NSUP + s
            if sync:
                pltpu.sync_copy(sidx_hbm.at[blk], sb_s[sb])
                pltpu.sync_copy(didx_hbm.at[blk], sb_d[sb])
                pltpu.sync_copy(w_hbm.at[blk], sb_w[sb])
            else:
                pltpu.async_copy(sidx_hbm.at[blk], sb_s[sb], stsem)
                pltpu.async_copy(didx_hbm.at[blk], sb_d[sb], stsem)
                pltpu.async_copy(w_hbm.at[blk], sb_w[sb], stsem)

        def _wait_stage(sb):
            pltpu.make_async_copy(sidx_hbm.at[0], sb_s[sb], stsem).wait()
            pltpu.make_async_copy(didx_hbm.at[0], sb_d[sb], stsem).wait()
            pltpu.make_async_copy(w_hbm.at[0], sb_w[sb], stsem).wait()

        def _issue(k, sb, b):
            pltpu.async_copy(table.at[sb_s[sb].at[k]], rows[b], gsem[b])

        def _wait_scat(b):
            pltpu.make_async_copy(
                rows[b], acc_sh.at[sb_d[0].at[0]], ssem[b]).wait()

        def _step(s, p, k):
            b = k % NBUF
            nxt = (k + 1) % NBUF

            if k <= 1 and p == 0:
                @pl.when(s >= 1)
                def _():
                    _wait_scat(nxt)
            else:
                _wait_scat(nxt)

            if k + 1 < SUP:
                _issue(k + 1, p, nxt)
            else:
                @pl.when(s + 1 < NSUP)
                def _():
                    _wait_stage(p ^ 1)
                    _issue(0, p ^ 1, nxt)

            pltpu.make_async_copy(
                table.at[sb_s[p].at[k]], rows[b], gsem[b]).wait()

            def _scale_grp(g, _):
                wv = sb_w[p][pl.ds(k * CHUNK + g * 16, 16)]
                for l in range(16):
                    wl = wv[l]
                    e = g * 16 + l
                    for j in range(D // 32, D // 16):
                        sl = pl.ds(j * 16, 16)
                        rows[b][e, sl] = rows[b][e, sl] * wl
                return 0
            lax.fori_loop(0, CHUNK // 16, _scale_grp, 0)
            pltpu.async_copy(rows[b], acc_sh.at[sb_d[p].at[k]], ssem[b],
                             add=True)

        _stage(0, 0, sync=True)
        _issue(0, 0, 0)

        def _super(h, _):
            for p in range(2):
                s = h * 2 + p

                @pl.when(s + 1 < NSUP)
                def _():
                    _stage(s + 1, p ^ 1, sync=False)
                for k in range(SUP):
                    _step(s, p, k)
            return 0
        lax.fori_loop(0, NSUP // 2, _super, 0)
        _wait_scat((NCH - 2) % NBUF)
        _wait_scat((NCH - 1) % NBUF)

    @pl.when(cid == 0)
    def _():
        _run(meff_hbm)

    @pl.when(cid == 1)
    def _():
        _run(mx_hbm)

    plsc.subcore_barrier()

    out_slice = pl.ds(tid * ROWS_PER_TILE, ROWS_PER_TILE)

    @pl.when(cid == 0)
    def _():
        pltpu.sync_copy(acc_sh.at[out_slice], dsum_hbm.at[out_slice])

    @pl.when(cid == 1)
    def _():
        pltpu.sync_copy(acc_sh.at[out_slice], hsum_hbm.at[out_slice])


def _segment_sums(meff, mx, sidx, didx, w):
    f32 = jnp.float32
    kern = pl.kernel(
        _sc_body,
        out_type=[
            jax.ShapeDtypeStruct((N_PAD, D), f32),
            jax.ShapeDtypeStruct((N_PAD, D), f32),
        ],
        mesh=plsc.VectorSubcoreMesh(core_axis_name="c", subcore_axis_name="s"),
        scratch_types=[
            pltpu.VMEM((SUP, CHUNK), jnp.int32),
            pltpu.VMEM((SUP, CHUNK), jnp.int32),
            pltpu.VMEM((SUP, CHUNK), jnp.int32),
            pltpu.VMEM((SUP, CHUNK), jnp.int32),
            pltpu.VMEM((SUP * CHUNK,), f32),
            pltpu.VMEM((SUP * CHUNK,), f32),
            pltpu.VMEM((CHUNK, D), f32),
            pltpu.VMEM((CHUNK, D), f32),
            pltpu.VMEM((CHUNK, D), f32),
            pltpu.VMEM_SHARED((N_PAD, D), f32),
            pltpu.SemaphoreType.DMA,
            pltpu.SemaphoreType.DMA,
            pltpu.SemaphoreType.DMA,
            pltpu.SemaphoreType.DMA,
            pltpu.SemaphoreType.DMA,
            pltpu.SemaphoreType.DMA,
            pltpu.SemaphoreType.DMA,
        ],
    )
    return kern(meff, mx, sidx, didx, w)


def _epi_body(a0_ref, a1_ref, w_ref, out_ref):
    a0 = a0_ref[...]
    a1 = a1_ref[...]
    d = jnp.concatenate([a0[:, :D // 2], a1[:, :D // 2]], 1)
    hs = jnp.concatenate([a0[:, D // 2:], a1[:, D // 2:]], 1)
    am = jnp.where(d == 0.0, 0.0, 1.0 / d)
    h = hs * am
    p = jnp.dot(h, w_ref[...], preferred_element_type=jnp.float32)
    out_ref[...] = jnp.where(p > 0.0, p, jnp.exp(p) - 1.0)


def _epilogue(dsum, hsum, w):
    return pl.pallas_call(
        _epi_body,
        grid=(N_PAD // EPI_BLK,),
        in_specs=[
            pl.BlockSpec((EPI_BLK, D), lambda i: (i, 0)),
            pl.BlockSpec((EPI_BLK, D), lambda i: (i, 0)),
            pl.BlockSpec((D, D), lambda i: (0, 0)),
        ],
        out_specs=pl.BlockSpec((EPI_BLK, D), lambda i: (i, 0)),
        out_shape=jax.ShapeDtypeStruct((N_PAD, D), jnp.float32),
    )(dsum, hsum, w)


@jax.jit
def kernel(x, edge_index, edge_weight, train_fts_id, W, M):
    src = edge_index[0].astype(jnp.int32)
    dst = edge_index[1].astype(jnp.int32)
    w = edge_weight.astype(jnp.float32)

    pad = E_PAD - E
    sidx = jnp.concatenate([src, jnp.zeros((pad,), jnp.int32)])
    didx = jnp.concatenate([dst, jnp.full((pad,), N, jnp.int32)])
    wpad = jnp.concatenate([w, jnp.zeros((pad,), jnp.float32)])
    sidx = sidx.reshape(NS * NSUP, SUP, CHUNK)
    didx = didx.reshape(NS * NSUP, SUP, CHUNK)
    wpad = wpad.reshape(NS * NSUP, SUP * CHUNK)

    ids = train_fts_id.astype(jnp.int32)
    ids_pad = jnp.concatenate(
        [ids, jnp.full((NT_PAD - ids.shape[0],), -1, jnp.int32)]
    ).reshape(NT_PAD // 128, 128)

    meff, mx = _prologue(ids_pad, M, x)
    dsum, hsum = _segment_sums(meff, mx, sidx, didx, wpad)
    out = _epilogue(dsum, hsum, W)
    return out[:N]


# speedup vs baseline: 1.6852x; 1.0023x over previous
import functools

import jax
import jax.numpy as jnp
from jax import lax
from jax.experimental import pallas as pl
from jax.experimental.pallas import tpu as pltpu
from jax.experimental.pallas import tpu_sc as plsc

N = 10000
E = 320000
D = 128

NC = 2
NS = 16
CHUNK = 112
SUP = 6
NBUF = 3
NCH = 180
NSUP = NCH // SUP
PER_TILE = NCH * CHUNK
E_PAD = PER_TILE * NS
ROWS_PER_TILE = 640
N_PAD = ROWS_PER_TILE * NS

PRO_BLK = 400
EPI_BLK = 400
NT_PAD = 1024


def _pro_body(ids_ref, m_ref, x_ref, meff_ref, mx_ref):
    base = pl.program_id(0) * PRO_BLK
    rows = base + lax.broadcasted_iota(jnp.int32, (PRO_BLK, 1), 0)
    ids = ids_ref[...]
    hit = jnp.zeros((PRO_BLK, 1), dtype=jnp.bool_)
    for j in range(NT_PAD // 128):
        hit = hit | jnp.any(rows == ids[j:j + 1, :], axis=1, keepdims=True)
    meff = jnp.where(hit, 1.0, jax.nn.sigmoid(m_ref[...]))
    mx = meff * x_ref[...]
    meff_ref[...] = jnp.concatenate([meff[:, :D // 2], mx[:, :D // 2]], 1)
    mx_ref[...] = jnp.concatenate([meff[:, D // 2:], mx[:, D // 2:]], 1)


def _prologue(train_ids_pad, m, x):
    return pl.pallas_call(
        _pro_body,
        grid=(N // PRO_BLK,),
        in_specs=[
            pl.BlockSpec((NT_PAD // 128, 128), lambda i: (0, 0)),
            pl.BlockSpec((PRO_BLK, D), lambda i: (i, 0)),
            pl.BlockSpec((PRO_BLK, D), lambda i: (i, 0)),
        ],
        out_specs=[
            pl.BlockSpec((PRO_BLK, D), lambda i: (i, 0)),
            pl.BlockSpec((PRO_BLK, D), lambda i: (i, 0)),
        ],
        out_shape=[
            jax.ShapeDtypeStruct((N, D), jnp.float32),
            jax.ShapeDtypeStruct((N, D), jnp.float32),
        ],
    )(train_ids_pad, m, x)


def _sc_body(meff_hbm, mx_hbm, sidx_hbm, didx_hbm, w_hbm, dsum_hbm, hsum_hbm,
             sb_s0, sb_s1, sb_d0, sb_d1, sb_w0, sb_w1,
             rows0, rows1, rows2, acc_sh,
             gsem0, gsem1, gsem2, ssem0, ssem1, ssem2, stsem):
    cid = lax.axis_index("c")
    tid = lax.axis_index("s")
    sb_s = (sb_s0, sb_s1)
    sb_d = (sb_d0, sb_d1)
    sb_w = (sb_w0, sb_w1)
    rows = (rows0, rows1, rows2)
    gsem = (gsem0, gsem1, gsem2)
    ssem = (ssem0, ssem1, ssem2)

    def _zero_row(i, _):
        for j in range(D // 16):
            rows0[i, pl.ds(j * 16, 16)] = jnp.zeros((16,), jnp.float32)
        return 0
    lax.fori_loop(0, CHUNK, _zero_row, 0)
    zbase = tid * ROWS_PER_TILE
    for k in range(ROWS_PER_TILE // CHUNK):
        pltpu.sync_copy(
            rows0, acc_sh.at[pl.ds(zbase + k * CHUNK, CHUNK)])
    rem = ROWS_PER_TILE % CHUNK
    if rem:
        pltpu.sync_copy(
            rows0.at[pl.ds(0, rem)],
            acc_sh.at[pl.ds(zbase + ROWS_PER_TILE - rem, rem)])
    plsc.subcore_barrier()

    def _run(table):
        def _stage(s, sb, sync):
            blk = tid * NSUP + s
            if sync:
                pltpu.sync_copy(sidx_hbm.at[blk], sb_s[sb])
                pltpu.sync_copy(didx_hbm.at[blk], sb_d[sb])
                pltpu.sync_copy(w_hbm.at[blk], sb_w[sb])
            else:
                pltpu.async_copy(sidx_hbm.at[blk], sb_s[sb], stsem)
                pltpu.async_copy(didx_hbm.at[blk], sb_d[sb], stsem)
                pltpu.async_copy(w_hbm.at[blk], sb_w[sb], stsem)

        def _wait_stage(sb):
            pltpu.make_async_copy(sidx_hbm.at[0], sb_s[sb], stsem).wait()
            pltpu.make_async_copy(didx_hbm.at[0], sb_d[sb], stsem).wait()
            pltpu.make_async_copy(w_hbm.at[0], sb_w[sb], stsem).wait()

        def _issue(k, sb, b):
            pltpu.async_copy(table.at[sb_s[sb].at[k]], rows[b], gsem[b])

        def _wait_scat(b):
            pltpu.make_async_copy(
                rows[b], acc_sh.at[sb_d[0].at[0]], ssem[b]).wait()

        def _step(s, p, k):
            b = k % NBUF
            nxt = (k + 1) % NBUF

            if k <= 1 and p == 0:
                @pl.when(s >= 1)
                def _():
                    _wait_scat(nxt)
            else:
                _wait_scat(nxt)

            if k + 1 < SUP:
                _issue(k + 1, p, nxt)
            else:
                @pl.when(s + 1 < NSUP)
                def _():
                    _wait_stage(p ^ 1)
                    _issue(0, p ^ 1, nxt)

            pltpu.make_async_copy(
                table.at[sb_s[p].at[k]], rows[b], gsem[b]).wait()

            def _scale_grp(g, _):
                wv = sb_w[p][pl.ds(k * CHUNK + g * 16, 16)]
                for l in range(16):
                    wl = wv[l]
                    e = g * 16 + l
                    for j in range(D // 32, D // 16):
                        sl = pl.ds(j * 16, 16)
                        rows[b][e, sl] = rows[b][e, sl] * wl
                return 0
            lax.fori_loop(0, CHUNK // 16, _scale_grp, 0)
            pltpu.async_copy(rows[b], acc_sh.at[sb_d[p].at[k]], ssem[b],
                             add=True)

        _stage(0, 0, sync=True)
        _issue(0, 0, 0)

        def _super(h, _):
            for p in range(2):
                s = h * 2 + p

                @pl.when(s + 1 < NSUP)
                def _():
                    _stage(s + 1, p ^ 1, sync=False)
                for k in range(SUP):
                    _step(s, p, k)
            return 0
        lax.fori_loop(0, NSUP // 2, _super, 0)
        _wait_scat((NCH - 2) % NBUF)
        _wait_scat((NCH - 1) % NBUF)

    @pl.when(cid == 0)
    def _():
        _run(meff_hbm)

    @pl.when(cid == 1)
    def _():
        _run(mx_hbm)

    plsc.subcore_barrier()

    out_slice = pl.ds(tid * ROWS_PER_TILE, ROWS_PER_TILE)

    @pl.when(cid == 0)
    def _():
        pltpu.sync_copy(acc_sh.at[out_slice], dsum_hbm.at[out_slice])

    @pl.when(cid == 1)
    def _():
        pltpu.sync_copy(acc_sh.at[out_slice], hsum_hbm.at[out_slice])


def _segment_sums(meff, mx, sidx, didx, w):
    f32 = jnp.float32
    kern = pl.kernel(
        _sc_body,
        out_type=[
            jax.ShapeDtypeStruct((N_PAD, D), f32),
            jax.ShapeDtypeStruct((N_PAD, D), f32),
        ],
        mesh=plsc.VectorSubcoreMesh(core_axis_name="c", subcore_axis_name="s"),
        scratch_types=[
            pltpu.VMEM((SUP, CHUNK), jnp.int32),
            pltpu.VMEM((SUP, CHUNK), jnp.int32),
            pltpu.VMEM((SUP, CHUNK), jnp.int32),
            pltpu.VMEM((SUP, CHUNK), jnp.int32),
            pltpu.VMEM((SUP * CHUNK,), f32),
            pltpu.VMEM((SUP * CHUNK,), f32),
            pltpu.VMEM((CHUNK, D), f32),
            pltpu.VMEM((CHUNK, D), f32),
            pltpu.VMEM((CHUNK, D), f32),
            pltpu.VMEM_SHARED((N_PAD, D), f32),
            pltpu.SemaphoreType.DMA,
            pltpu.SemaphoreType.DMA,
            pltpu.SemaphoreType.DMA,
            pltpu.SemaphoreType.DMA,
            pltpu.SemaphoreType.DMA,
            pltpu.SemaphoreType.DMA,
            pltpu.SemaphoreType.DMA,
        ],
    )
    return kern(meff, mx, sidx, didx, w)


def _epi_body(a0_ref, a1_ref, w_ref, out_ref):
    a0 = a0_ref[...]
    a1 = a1_ref[...]
    d = jnp.concatenate([a0[:, :D // 2], a1[:, :D // 2]], 1)
    hs = jnp.concatenate([a0[:, D // 2:], a1[:, D // 2:]], 1)
    am = jnp.where(d == 0.0, 0.0, 1.0 / d)
    h = hs * am
    p = jnp.dot(h, w_ref[...], preferred_element_type=jnp.float32)
    out_ref[...] = jnp.where(p > 0.0, p, jnp.exp(p) - 1.0)


def _epilogue(dsum, hsum, w):
    return pl.pallas_call(
        _epi_body,
        grid=(N // EPI_BLK,),
        in_specs=[
            pl.BlockSpec((EPI_BLK, D), lambda i: (i, 0)),
            pl.BlockSpec((EPI_BLK, D), lambda i: (i, 0)),
            pl.BlockSpec((D, D), lambda i: (0, 0)),
        ],
        out_specs=pl.BlockSpec((EPI_BLK, D), lambda i: (i, 0)),
        out_shape=jax.ShapeDtypeStruct((N, D), jnp.float32),
    )(dsum, hsum, w)


@jax.jit
def kernel(x, edge_index, edge_weight, train_fts_id, W, M):
    src = edge_index[0].astype(jnp.int32)
    dst = edge_index[1].astype(jnp.int32)
    w = edge_weight.astype(jnp.float32)

    pad = E_PAD - E
    sidx = jnp.concatenate([src, jnp.zeros((pad,), jnp.int32)])
    didx = jnp.concatenate([dst, jnp.full((pad,), N, jnp.int32)])
    wpad = jnp.concatenate([w, jnp.zeros((pad,), jnp.float32)])
    sidx = sidx.reshape(NS * NSUP, SUP, CHUNK)
    didx = didx.reshape(NS * NSUP, SUP, CHUNK)
    wpad = wpad.reshape(NS * NSUP, SUP * CHUNK)

    ids = train_fts_id.astype(jnp.int32)
    ids_pad = jnp.concatenate(
        [ids, jnp.full((NT_PAD - ids.shape[0],), -1, jnp.int32)]
    ).reshape(NT_PAD // 128, 128)

    meff, mx = _prologue(ids_pad, M, x)
    dsum, hsum = _segment_sums(meff, mx, sidx, didx, wpad)
    return _epilogue(dsum, hsum, W)
